# baseline (speedup 1.0000x reference)
import functools

import jax
import jax.numpy as jnp
import numpy as np
from jax import lax
from jax.experimental import pallas as pl
from jax.experimental.pallas import tpu as pltpu
from jax.experimental.pallas import tpu_sc as plsc

_PI = float(np.pi)
_DEPTH0, _DEPTH1 = 28.01, 16.32
_DIMS_TBL = ((3.88, 1.63, 1.53), (1.76, 1.73, 0.6), (0.84, 1.76, 0.66))
_THR = 0.25
_K = 100
_KPAD = 128

_B, _C, _H, _W = 8, 3, 192, 640
_HW = _H * _W
_NSEG = _C * _H * (_W // 128)
_PMAX = 256
_TARGET = 128
_QVAL = float(np.nextafter(np.float32(0.25), np.float32(1.0)))

_SX = (-1.0, 1.0, 1.0, 1.0, 1.0, -1.0, -1.0, -1.0)
_SY = (-1.0, -1.0, 0.0, 0.0, -1.0, -1.0, 0.0, 0.0)
_SZ = (-1.0, -1.0, -1.0, 1.0, 1.0, 1.0, 1.0, -1.0)


def _nms_body(h_ref, z_ref, m_ref):
    x = h_ref[0]
    ncol = jnp.full((_H, 1), -1.0, jnp.float32)
    left = jnp.concatenate([x[:, 1:], ncol], axis=1)
    right = jnp.concatenate([ncol, x[:, :-1]], axis=1)
    rm = jnp.maximum(jnp.maximum(left, right), x)
    nrow = jnp.full((1, _W), -1.0, jnp.float32)
    up = jnp.concatenate([rm[1:, :], nrow], axis=0)
    dn = jnp.concatenate([nrow, rm[:-1, :]], axis=0)
    hm = jnp.maximum(jnp.maximum(up, dn), rm)
    z = jnp.where(hm == x, x, 0.0)
    z_ref[0] = z
    cols = [jnp.max(z[:, i * 128:(i + 1) * 128], axis=1, keepdims=True)
            for i in range(_W // 128)]
    m_ref[0] = jnp.concatenate(cols, axis=1)


def _nms_call(heat, interpret=False):
    return pl.pallas_call(
        _nms_body,
        grid=(_B * _C,),
        in_specs=[pl.BlockSpec((1, _H, _W), lambda i: (i, 0, 0))],
        out_specs=[
            pl.BlockSpec((1, _H, _W), lambda i: (i, 0, 0)),
            pl.BlockSpec((1, _H, _W // 128), lambda i: (i, 0, 0)),
        ],
        out_shape=[
            jax.ShapeDtypeStruct((_B * _C, _H, _W), jnp.float32),
            jax.ShapeDtypeStruct((_B * _C, _H, _W // 128), jnp.float32),
        ],
        interpret=interpret,
    )(heat)


def _sc_body(z_hbm, m_hbm, reg_hbm, sc_out, fl_out, po_out,
             m_v, segid_v, cmax_v, seg_v, os_v, of_v, gi_v, pv_v, sem):
    nc = 2
    wid = lax.axis_index("s") * nc + lax.axis_index("c")

    @pl.when(wid < _B)
    def _worker():
        b = wid
        pltpu.sync_copy(m_hbm.at[b], m_v)

        zero16i = jnp.zeros((16,), jnp.int32)
        zero16f = jnp.zeros((16,), jnp.float32)
        neg16 = jnp.full((16,), -1.0, jnp.float32)
        lane = lax.iota(jnp.int32, 16)

        def init_body(c, _):
            cmax_v[pl.ds(c * 16, 16)] = neg16
            segid_v[pl.ds(c * 16, 16)] = zero16i
            return 0

        lax.fori_loop(0, _PMAX // 16, init_body, 0)

        def init2_body(c, _):
            os_v[pl.ds(c * 16, 16)] = zero16f
            of_v[pl.ds(c * 16, 16)] = zero16i
            return 0

        lax.fori_loop(0, _KPAD // 16, init2_body, 0)

        nchunk = _NSEG // 16

        def _count_ge(t):
            def body(c, acc):
                for u in range(10):
                    v = m_v[pl.ds(c * 160 + u * 16, 16)]
                    acc = acc + jnp.where(v >= t, 1, 0)
                return acc
            accv = lax.fori_loop(0, nchunk // 10, body, zero16i)
            return jnp.sum(accv)

        def bis_cond(st):
            return (st[0] < 18) & jnp.logical_not(st[3])

        def bis_body(st):
            i, lo, hi, _ = st
            mid = (lo + hi) * 0.5
            cnt = _count_ge(mid)
            good = (cnt >= _K) & (cnt <= _PMAX - 16)
            big = cnt >= _TARGET
            lo2 = jnp.where(big | good, mid, lo)
            hi2 = jnp.where(big, hi, mid)
            return (i + 1, lo2, hi2, good)

        _, lo, hi, _ = lax.while_loop(
            bis_cond, bis_body,
            (jnp.int32(0), jnp.float32(_QVAL), jnp.float32(1.0),
             jnp.bool_(False)))
        tval = lo

        def comp_body(c, cur):
            v = m_v[pl.ds(c * 16, 16)]
            msk = v >= tval
            n = jnp.max(plsc.all_reduce_population_count(msk))

            @pl.when(cur <= _PMAX - 16)
            def _():
                ids = lane + c * 16
                plsc.store_compressed(segid_v.at[pl.ds(cur, 16)], ids,
                                      mask=msk)
                plsc.store_compressed(cmax_v.at[pl.ds(cur, 16)], v, mask=msk)

            return jnp.where(cur <= _PMAX - 16, cur + n, cur)

        ncand = lax.fori_loop(0, nchunk, comp_body, jnp.int32(0))

        base_row = b * _NSEG

        def gidx_body(c, _):
            gi_v[c // 8, pl.ds((c % 8) * 16, 16)] = (
                segid_v[pl.ds(c * 16, 16)] + base_row)
            return 0

        lax.fori_loop(0, _PMAX // 16, gidx_body, 0)
        cp0 = pltpu.async_copy(z_hbm.at[gi_v.at[0]], seg_v.at[pl.ds(0, 128)],
                               sem)

        @pl.when(ncand > 128)
        def _extra_gather():
            pltpu.async_copy(z_hbm.at[gi_v.at[1]],
                             seg_v.at[pl.ds(128, 128)], sem).wait()

        cp0.wait()

        def _extract_i(ref, i):
            chunk = ref[pl.ds((i // 16) * 16, 16)]
            return jnp.max(jnp.where(lane == (i % 16), chunk, 0))

        def hier_body(c, h):
            return jnp.where(lane == c, jnp.max(cmax_v[pl.ds(c * 16, 16)]), h)

        c2_init = lax.fori_loop(0, _PMAX // 16, hier_body, neg16)

        def pop_body(k, c2):
            m = jnp.max(c2)
            c = jnp.min(jnp.where(c2 == m, lane, 9999))
            v = cmax_v[pl.ds(c * 16, 16)]
            sl = jnp.min(jnp.where(v == m, lane, 9999))
            si = c * 16 + sl

            jv = jnp.full((16,), 9999, jnp.int32)
            for u in range(8):
                sv = seg_v[si, pl.ds(u * 16, 16)]
                jv = jnp.minimum(jv, jnp.where(sv == m, lane + u * 16, 9999))
            j = jnp.minimum(jnp.min(jv), 127)
            sid = _extract_i(segid_v, si)
            flat = sid * 128 + j
            kc = (k // 16) * 16
            kl = k % 16
            os_v[pl.ds(kc, 16)] = jnp.where(lane == kl, m,
                                            os_v[pl.ds(kc, 16)])
            of_v[pl.ds(kc, 16)] = jnp.where(lane == kl, flat,
                                            of_v[pl.ds(kc, 16)])
            jc = j // 16
            jl = j % 16
            nms = neg16
            for u in range(8):
                sv = seg_v[si, pl.ds(u * 16, 16)]
                sv = jnp.where((jc == u) & (lane == jl), -1.0, sv)
                nms = jnp.maximum(nms, sv)
            nm = jnp.max(nms)
            seg_v[si, pl.ds(jc * 16, 16)] = jnp.where(
                lane == jl, -1.0, seg_v[si, pl.ds(jc * 16, 16)])
            nv = jnp.where(lane == sl, nm, v)
            cmax_v[pl.ds(c * 16, 16)] = nv
            return jnp.where(lane == c, jnp.max(nv), c2)

        lax.fori_loop(0, _K, pop_body, c2_init)

        rbase = b * (8 * _HW)

        def ridx_body(p, _):
            r = p // 8
            c = p % 8
            sp = of_v[pl.ds(c * 16, 16)] % _HW
            gi_v[r, pl.ds(c * 16, 16)] = sp + (rbase + r * _HW)
            return 0

        lax.fori_loop(0, 64, ridx_body, 0)
        cps = [pltpu.async_copy(reg_hbm.at[gi_v.at[r]], pv_v.at[r], sem)
               for r in range(8)]
        for cp in cps:
            cp.wait()

        pltpu.sync_copy(os_v, sc_out.at[b])
        pltpu.sync_copy(of_v, fl_out.at[b])
        pltpu.sync_copy(pv_v, po_out.at[b])


@functools.lru_cache(maxsize=None)
def _get_sc_select():
  return functools.partial(
    pl.kernel,
    mesh=plsc.VectorSubcoreMesh(core_axis_name="c", subcore_axis_name="s"),
    compiler_params=pltpu.CompilerParams(needs_layout_passes=False),
    out_type=[
        jax.ShapeDtypeStruct((_B, _KPAD), jnp.float32),
        jax.ShapeDtypeStruct((_B, _KPAD), jnp.int32),
        jax.ShapeDtypeStruct((_B, 8, _KPAD), jnp.float32),
    ],
    scratch_types=[
        pltpu.VMEM((_NSEG,), jnp.float32),
        pltpu.VMEM((_PMAX,), jnp.int32),
        pltpu.VMEM((_PMAX,), jnp.float32),
        pltpu.VMEM((_PMAX, 128), jnp.float32),
        pltpu.VMEM((_KPAD,), jnp.float32),
        pltpu.VMEM((_KPAD,), jnp.int32),
        pltpu.VMEM((8, 128), jnp.int32),
        pltpu.VMEM((8, 128), jnp.float32),
        pltpu.SemaphoreType.DMA,
    ],
  )(_sc_body)


def _inv3(m9):
    a, bb, cc = m9[:, 0:1], m9[:, 1:2], m9[:, 2:3]
    d, e, f = m9[:, 3:4], m9[:, 4:5], m9[:, 5:6]
    g, h, i = m9[:, 6:7], m9[:, 7:8], m9[:, 8:9]
    A = e * i - f * h
    Bc = -(d * i - f * g)
    Cc = d * h - e * g
    det = a * A + bb * Bc + cc * Cc
    r = 1.0 / det
    return ((A * r, -(bb * i - cc * h) * r, (bb * f - cc * e) * r),
            (Bc * r, (a * i - cc * g) * r, -(a * f - cc * d) * r),
            (Cc * r, -(a * h - bb * g) * r, (a * e - bb * d) * r))


def _decode_body(s_ref, f_ref, p0r, p1r, p2r, p3r, p4r, p5r, p6r, p7r,
                 t_ref, k_ref, i_ref, o_ref):
    score = s_ref[...]
    flat = f_ref[...]
    cls = flat // _HW
    sp = flat - cls * _HW
    ysi = sp // _W
    xsi = sp - ysi * _W
    xs = xsi.astype(jnp.float32)
    ys = ysi.astype(jnp.float32)
    clsf = cls.astype(jnp.float32)

    depth = p0r[...] * _DEPTH1 + _DEPTH0
    px = xs + p1r[...]
    py = ys + p2r[...]

    ti = _inv3(t_ref[...])
    ix = (ti[0][0] * px + ti[0][1] * py + ti[0][2]) * depth
    iy = (ti[1][0] * px + ti[1][1] * py + ti[1][2]) * depth
    iz = (ti[2][0] * px + ti[2][1] * py + ti[2][2]) * depth
    ki = _inv3(k_ref[...])
    lx = ki[0][0] * ix + ki[0][1] * iy + ki[0][2] * iz
    ly = ki[1][0] * ix + ki[1][1] * iy + ki[1][2] * iz
    lz = ki[2][0] * ix + ki[2][1] * iy + ki[2][2] * iz

    is0 = jnp.where(cls == 0, 1.0, 0.0)
    is1 = jnp.where(cls == 1, 1.0, 0.0)
    is2 = 1.0 - is0 - is1
    d0 = jnp.exp(p3r[...]) * (is0 * _DIMS_TBL[0][0] + is1 * _DIMS_TBL[1][0]
                              + is2 * _DIMS_TBL[2][0])
    d1 = jnp.exp(p4r[...]) * (is0 * _DIMS_TBL[0][1] + is1 * _DIMS_TBL[1][1]
                              + is2 * _DIMS_TBL[2][1])
    d2 = jnp.exp(p5r[...]) * (is0 * _DIMS_TBL[0][2] + is1 * _DIMS_TBL[1][2]
                              + is2 * _DIMS_TBL[2][2])
    ly = ly + d1 * 0.5

    one = jnp.ones_like(lx)
    rays = jnp.arctan2(lx / (lz + 1e-7), one)
    ori0, ori1 = p6r[...], p7r[...]
    a0 = jnp.arctan2(ori0 / (ori1 + 1e-7), one)
    alpha = jnp.where(ori1 >= 0, a0 - _PI / 2.0, a0 + _PI / 2.0)
    roty = alpha + rays
    roty = jnp.where(roty > _PI, roty - 2.0 * _PI, roty)
    roty = jnp.where(roty < -_PI, roty + 2.0 * _PI, roty)

    cr = jnp.cos(roty)
    sr = jnp.sin(roty)
    kk = k_ref[...]
    k00, k01, k02 = kk[:, 0:1], kk[:, 1:2], kk[:, 2:3]
    k10, k11, k12 = kk[:, 3:4], kk[:, 4:5], kk[:, 5:6]
    k20, k21, k22 = kk[:, 6:7], kk[:, 7:8], kk[:, 8:9]

    big = jnp.float32(1e30)
    umin = jnp.full_like(score, big)
    umax = jnp.full_like(score, -big)
    vmin = jnp.full_like(score, big)
    vmax = jnp.full_like(score, -big)
    for t in range(8):
        cx = d0 * (0.5 * _SX[t])
        cy = d1 * _SY[t]
        cz = d2 * (0.5 * _SZ[t])
        X = cr * cx + sr * cz + lx
        Y = cy + ly
        Zc = -sr * cx + cr * cz + lz
        w_ = k20 * X + k21 * Y + k22 * Zc
        u_ = (k00 * X + k01 * Y + k02 * Zc) / w_
        v_ = (k10 * X + k11 * Y + k12 * Zc) / w_
        umin = jnp.minimum(umin, u_)
        umax = jnp.maximum(umax, u_)
        vmin = jnp.minimum(vmin, v_)
        vmax = jnp.maximum(vmax, v_)

    iw = i_ref[0:1, 0:1]
    ih = i_ref[0:1, 1:2]
    xmin = jnp.clip(umin, 0.0, iw)
    xmax = jnp.clip(umax, 0.0, iw)
    ymin = jnp.clip(vmin, 0.0, ih)
    ymax = jnp.clip(vmax, 0.0, ih)

    keep = score > _THR
    rows = (clsf, alpha, xmin, ymin, xmax, ymax, d1, d2, d0,
            lx, ly, lz, roty, score)
    for idx, rr in enumerate(rows):
        o_ref[idx] = jnp.where(keep, rr, 0.0)


def _decode_call(sc, fl, pois, t9, k9, isz, interpret=False):
    full = lambda s: pl.BlockSpec(s, lambda: tuple(0 for _ in s))
    return pl.pallas_call(
        _decode_body,
        in_specs=[full((_B, _KPAD)), full((_B, _KPAD))]
        + [full((_B, _KPAD))] * 8
        + [full((_B, 9)), full((_B, 9)), full((_B, 2))],
        out_specs=[full((14, _B, _KPAD))],
        out_shape=[jax.ShapeDtypeStruct((14, _B, _KPAD), jnp.float32)],
        interpret=interpret,
    )(sc, fl, *pois, t9, k9, isz)


def kernel(pred_heatmap, pred_regression, trans_mat, Kmat, img_size):
    heat = pred_heatmap.reshape(_B * _C, _H, _W)
    z, m = _nms_call(heat)
    sc, fl, po = _get_sc_select()(z.reshape(_B * _NSEG, 128),
                                  m.reshape(_B, _NSEG),
                                  pred_regression.reshape(-1))
    pois = [po[:, r, :] for r in range(8)]
    out14 = _decode_call(sc, fl, pois, trans_mat.reshape(_B, 9),
                         Kmat.reshape(_B, 9),
                         img_size.astype(jnp.float32))[0]
    return jnp.transpose(out14, (1, 2, 0))[:, :_K, :].reshape(_B * _K, 14)


# speedup vs baseline: 1.0513x; 1.0513x over previous
import functools

import jax
import jax.numpy as jnp
import numpy as np
from jax import lax
from jax.experimental import pallas as pl
from jax.experimental.pallas import tpu as pltpu
from jax.experimental.pallas import tpu_sc as plsc

_PI = float(np.pi)
_DEPTH0, _DEPTH1 = 28.01, 16.32
_DIMS_TBL = ((3.88, 1.63, 1.53), (1.76, 1.73, 0.6), (0.84, 1.76, 0.66))
_THR = 0.25
_K = 100
_KPAD = 128

_B, _C, _H, _W = 8, 3, 192, 640
_HW = _H * _W
_NSEG = _C * _H * (_W // 128)
_PMAX = 256
_TARGET = 128
_QVAL = float(np.nextafter(np.float32(0.25), np.float32(1.0)))

_SX = (-1.0, 1.0, 1.0, 1.0, 1.0, -1.0, -1.0, -1.0)
_SY = (-1.0, -1.0, 0.0, 0.0, -1.0, -1.0, 0.0, 0.0)
_SZ = (-1.0, -1.0, -1.0, 1.0, 1.0, 1.0, 1.0, -1.0)


def _nms_body(h_ref, z_ref, m_ref):
    x = h_ref[0]
    ncol = jnp.full((_H, 1), -1.0, jnp.float32)
    left = jnp.concatenate([x[:, 1:], ncol], axis=1)
    right = jnp.concatenate([ncol, x[:, :-1]], axis=1)
    rm = jnp.maximum(jnp.maximum(left, right), x)
    nrow = jnp.full((1, _W), -1.0, jnp.float32)
    up = jnp.concatenate([rm[1:, :], nrow], axis=0)
    dn = jnp.concatenate([nrow, rm[:-1, :]], axis=0)
    hm = jnp.maximum(jnp.maximum(up, dn), rm)
    z = jnp.where(hm == x, x, 0.0)
    z_ref[0] = z
    cols = [jnp.max(z[:, i * 128:(i + 1) * 128], axis=1, keepdims=True)
            for i in range(_W // 128)]
    m_ref[0] = jnp.concatenate(cols, axis=1)


def _nms_call(heat, interpret=False):
    return pl.pallas_call(
        _nms_body,
        grid=(_B * _C,),
        compiler_params=pltpu.CompilerParams(
            dimension_semantics=("parallel",)),
        in_specs=[pl.BlockSpec((1, _H, _W), lambda i: (i, 0, 0))],
        out_specs=[
            pl.BlockSpec((1, _H, _W), lambda i: (i, 0, 0)),
            pl.BlockSpec((1, _H, _W // 128), lambda i: (i, 0, 0)),
        ],
        out_shape=[
            jax.ShapeDtypeStruct((_B * _C, _H, _W), jnp.float32),
            jax.ShapeDtypeStruct((_B * _C, _H, _W // 128), jnp.float32),
        ],
        interpret=interpret,
    )(heat)


def _sc_body(z_hbm, m_hbm, reg_hbm, sc_out, fl_out, po_out,
             m_v, segid_v, cmax_v, seg_v, os_v, of_v, gi_v, pv_v, sem):
    nc = 2
    wid = lax.axis_index("s") * nc + lax.axis_index("c")

    @pl.when(wid < _B)
    def _worker():
        b = wid
        pltpu.sync_copy(m_hbm.at[b], m_v)

        zero16i = jnp.zeros((16,), jnp.int32)
        zero16f = jnp.zeros((16,), jnp.float32)
        neg16 = jnp.full((16,), -1.0, jnp.float32)
        lane = lax.iota(jnp.int32, 16)

        def init_body(c, _):
            cmax_v[pl.ds(c * 16, 16)] = neg16
            segid_v[pl.ds(c * 16, 16)] = zero16i
            return 0

        lax.fori_loop(0, _PMAX // 16, init_body, 0)

        def init2_body(c, _):
            os_v[pl.ds(c * 16, 16)] = zero16f
            of_v[pl.ds(c * 16, 16)] = zero16i
            return 0

        lax.fori_loop(0, _KPAD // 16, init2_body, 0)

        nchunk = _NSEG // 16

        def _count_ge(t):
            def body(c, acc):
                for u in range(10):
                    v = m_v[pl.ds(c * 160 + u * 16, 16)]
                    acc = acc + jnp.where(v >= t, 1, 0)
                return acc
            accv = lax.fori_loop(0, nchunk // 10, body, zero16i)
            return jnp.sum(accv)

        def bis(_, lohi):
            lo, hi = lohi
            mid = (lo + hi) * 0.5
            cnt = _count_ge(mid)
            big = cnt >= _TARGET
            return jnp.where(big, mid, lo), jnp.where(big, hi, mid)

        lo, hi = lax.fori_loop(0, 18, bis, (jnp.float32(_QVAL),
                                            jnp.float32(1.0)))
        tval = lo

        def comp_body(c, cur):
            v = m_v[pl.ds(c * 16, 16)]
            msk = v >= tval
            n = jnp.max(plsc.all_reduce_population_count(msk))

            @pl.when(cur <= _PMAX - 16)
            def _():
                ids = lane + c * 16
                plsc.store_compressed(segid_v.at[pl.ds(cur, 16)], ids,
                                      mask=msk)
                plsc.store_compressed(cmax_v.at[pl.ds(cur, 16)], v, mask=msk)

            return jnp.where(cur <= _PMAX - 16, cur + n, cur)

        ncand = lax.fori_loop(0, nchunk, comp_body, jnp.int32(0))

        base_row = b * _NSEG

        def gidx_body(c, _):
            gi_v[c // 8, pl.ds((c % 8) * 16, 16)] = (
                segid_v[pl.ds(c * 16, 16)] + base_row)
            return 0

        lax.fori_loop(0, _PMAX // 16, gidx_body, 0)
        cp0 = pltpu.async_copy(z_hbm.at[gi_v.at[0]], seg_v.at[pl.ds(0, 128)],
                               sem)

        @pl.when(ncand > 128)
        def _extra_gather():
            pltpu.async_copy(z_hbm.at[gi_v.at[1]],
                             seg_v.at[pl.ds(128, 128)], sem).wait()

        cp0.wait()

        def _extract_i(ref, i):
            chunk = ref[pl.ds((i // 16) * 16, 16)]
            return jnp.max(jnp.where(lane == (i % 16), chunk, 0))

        def hier_body(c, h):
            return jnp.where(lane == c, jnp.max(cmax_v[pl.ds(c * 16, 16)]), h)

        c2_init = lax.fori_loop(0, _PMAX // 16, hier_body, neg16)

        def pop_body(k, c2):
            m = jnp.max(c2)
            c = jnp.min(jnp.where(c2 == m, lane, 9999))
            v = cmax_v[pl.ds(c * 16, 16)]
            sl = jnp.min(jnp.where(v == m, lane, 9999))
            si = c * 16 + sl

            jv = jnp.full((16,), 9999, jnp.int32)
            for u in range(8):
                sv = seg_v[si, pl.ds(u * 16, 16)]
                jv = jnp.minimum(jv, jnp.where(sv == m, lane + u * 16, 9999))
            j = jnp.minimum(jnp.min(jv), 127)
            sid = _extract_i(segid_v, si)
            flat = sid * 128 + j
            kc = (k // 16) * 16
            kl = k % 16
            os_v[pl.ds(kc, 16)] = jnp.where(lane == kl, m,
                                            os_v[pl.ds(kc, 16)])
            of_v[pl.ds(kc, 16)] = jnp.where(lane == kl, flat,
                                            of_v[pl.ds(kc, 16)])
            jc = j // 16
            jl = j % 16
            nms = neg16
            for u in range(8):
                sv = seg_v[si, pl.ds(u * 16, 16)]
                sv = jnp.where((jc == u) & (lane == jl), -1.0, sv)
                nms = jnp.maximum(nms, sv)
            nm = jnp.max(nms)
            seg_v[si, pl.ds(jc * 16, 16)] = jnp.where(
                lane == jl, -1.0, seg_v[si, pl.ds(jc * 16, 16)])
            nv = jnp.where(lane == sl, nm, v)
            cmax_v[pl.ds(c * 16, 16)] = nv
            return jnp.where(lane == c, jnp.max(nv), c2)

        lax.fori_loop(0, _K, pop_body, c2_init)

        rbase = b * (8 * _HW)

        def ridx_body(p, _):
            r = p // 8
            c = p % 8
            sp = of_v[pl.ds(c * 16, 16)] % _HW
            gi_v[r, pl.ds(c * 16, 16)] = sp + (rbase + r * _HW)
            return 0

        lax.fori_loop(0, 64, ridx_body, 0)
        cps = [pltpu.async_copy(reg_hbm.at[gi_v.at[r]], pv_v.at[r], sem)
               for r in range(8)]
        for cp in cps:
            cp.wait()

        pltpu.sync_copy(os_v, sc_out.at[b])
        pltpu.sync_copy(of_v, fl_out.at[b])
        pltpu.sync_copy(pv_v, po_out.at[b])


@functools.lru_cache(maxsize=None)
def _get_sc_select():
  return functools.partial(
    pl.kernel,
    mesh=plsc.VectorSubcoreMesh(core_axis_name="c", subcore_axis_name="s"),
    compiler_params=pltpu.CompilerParams(needs_layout_passes=False),
    out_type=[
        jax.ShapeDtypeStruct((_B, _KPAD), jnp.float32),
        jax.ShapeDtypeStruct((_B, _KPAD), jnp.int32),
        jax.ShapeDtypeStruct((_B, 8, _KPAD), jnp.float32),
    ],
    scratch_types=[
        pltpu.VMEM((_NSEG,), jnp.float32),
        pltpu.VMEM((_PMAX,), jnp.int32),
        pltpu.VMEM((_PMAX,), jnp.float32),
        pltpu.VMEM((_PMAX, 128), jnp.float32),
        pltpu.VMEM((_KPAD,), jnp.float32),
        pltpu.VMEM((_KPAD,), jnp.int32),
        pltpu.VMEM((8, 128), jnp.int32),
        pltpu.VMEM((8, 128), jnp.float32),
        pltpu.SemaphoreType.DMA,
    ],
  )(_sc_body)


def _inv3(m9):
    a, bb, cc = m9[:, 0:1], m9[:, 1:2], m9[:, 2:3]
    d, e, f = m9[:, 3:4], m9[:, 4:5], m9[:, 5:6]
    g, h, i = m9[:, 6:7], m9[:, 7:8], m9[:, 8:9]
    A = e * i - f * h
    Bc = -(d * i - f * g)
    Cc = d * h - e * g
    det = a * A + bb * Bc + cc * Cc
    r = 1.0 / det
    return ((A * r, -(bb * i - cc * h) * r, (bb * f - cc * e) * r),
            (Bc * r, (a * i - cc * g) * r, -(a * f - cc * d) * r),
            (Cc * r, -(a * h - bb * g) * r, (a * e - bb * d) * r))


def _decode_body(s_ref, f_ref, p0r, p1r, p2r, p3r, p4r, p5r, p6r, p7r,
                 t_ref, k_ref, i_ref, o_ref):
    score = s_ref[...]
    flat = f_ref[...]
    cls = flat // _HW
    sp = flat - cls * _HW
    ysi = sp // _W
    xsi = sp - ysi * _W
    xs = xsi.astype(jnp.float32)
    ys = ysi.astype(jnp.float32)
    clsf = cls.astype(jnp.float32)

    depth = p0r[...] * _DEPTH1 + _DEPTH0
    px = xs + p1r[...]
    py = ys + p2r[...]

    ti = _inv3(t_ref[...])
    ix = (ti[0][0] * px + ti[0][1] * py + ti[0][2]) * depth
    iy = (ti[1][0] * px + ti[1][1] * py + ti[1][2]) * depth
    iz = (ti[2][0] * px + ti[2][1] * py + ti[2][2]) * depth
    ki = _inv3(k_ref[...])
    lx = ki[0][0] * ix + ki[0][1] * iy + ki[0][2] * iz
    ly = ki[1][0] * ix + ki[1][1] * iy + ki[1][2] * iz
    lz = ki[2][0] * ix + ki[2][1] * iy + ki[2][2] * iz

    is0 = jnp.where(cls == 0, 1.0, 0.0)
    is1 = jnp.where(cls == 1, 1.0, 0.0)
    is2 = 1.0 - is0 - is1
    d0 = jnp.exp(p3r[...]) * (is0 * _DIMS_TBL[0][0] + is1 * _DIMS_TBL[1][0]
                              + is2 * _DIMS_TBL[2][0])
    d1 = jnp.exp(p4r[...]) * (is0 * _DIMS_TBL[0][1] + is1 * _DIMS_TBL[1][1]
                              + is2 * _DIMS_TBL[2][1])
    d2 = jnp.exp(p5r[...]) * (is0 * _DIMS_TBL[0][2] + is1 * _DIMS_TBL[1][2]
                              + is2 * _DIMS_TBL[2][2])
    ly = ly + d1 * 0.5

    one = jnp.ones_like(lx)
    rays = jnp.arctan2(lx / (lz + 1e-7), one)
    ori0, ori1 = p6r[...], p7r[...]
    a0 = jnp.arctan2(ori0 / (ori1 + 1e-7), one)
    alpha = jnp.where(ori1 >= 0, a0 - _PI / 2.0, a0 + _PI / 2.0)
    roty = alpha + rays
    roty = jnp.where(roty > _PI, roty - 2.0 * _PI, roty)
    roty = jnp.where(roty < -_PI, roty + 2.0 * _PI, roty)

    cr = jnp.cos(roty)
    sr = jnp.sin(roty)
    kk = k_ref[...]
    k00, k01, k02 = kk[:, 0:1], kk[:, 1:2], kk[:, 2:3]
    k10, k11, k12 = kk[:, 3:4], kk[:, 4:5], kk[:, 5:6]
    k20, k21, k22 = kk[:, 6:7], kk[:, 7:8], kk[:, 8:9]

    big = jnp.float32(1e30)
    umin = jnp.full_like(score, big)
    umax = jnp.full_like(score, -big)
    vmin = jnp.full_like(score, big)
    vmax = jnp.full_like(score, -big)
    for t in range(8):
        cx = d0 * (0.5 * _SX[t])
        cy = d1 * _SY[t]
        cz = d2 * (0.5 * _SZ[t])
        X = cr * cx + sr * cz + lx
        Y = cy + ly
        Zc = -sr * cx + cr * cz + lz
        w_ = k20 * X + k21 * Y + k22 * Zc
        u_ = (k00 * X + k01 * Y + k02 * Zc) / w_
        v_ = (k10 * X + k11 * Y + k12 * Zc) / w_
        umin = jnp.minimum(umin, u_)
        umax = jnp.maximum(umax, u_)
        vmin = jnp.minimum(vmin, v_)
        vmax = jnp.maximum(vmax, v_)

    iw = i_ref[0:1, 0:1]
    ih = i_ref[0:1, 1:2]
    xmin = jnp.clip(umin, 0.0, iw)
    xmax = jnp.clip(umax, 0.0, iw)
    ymin = jnp.clip(vmin, 0.0, ih)
    ymax = jnp.clip(vmax, 0.0, ih)

    keep = score > _THR
    rows = (clsf, alpha, xmin, ymin, xmax, ymax, d1, d2, d0,
            lx, ly, lz, roty, score)
    for idx, rr in enumerate(rows):
        o_ref[idx] = jnp.where(keep, rr, 0.0)


def _decode_call(sc, fl, pois, t9, k9, isz, interpret=False):
    full = lambda s: pl.BlockSpec(s, lambda: tuple(0 for _ in s))
    return pl.pallas_call(
        _decode_body,
        in_specs=[full((_B, _KPAD)), full((_B, _KPAD))]
        + [full((_B, _KPAD))] * 8
        + [full((_B, 9)), full((_B, 9)), full((_B, 2))],
        out_specs=[full((14, _B, _KPAD))],
        out_shape=[jax.ShapeDtypeStruct((14, _B, _KPAD), jnp.float32)],
        interpret=interpret,
    )(sc, fl, *pois, t9, k9, isz)


def kernel(pred_heatmap, pred_regression, trans_mat, Kmat, img_size):
    heat = pred_heatmap.reshape(_B * _C, _H, _W)
    z, m = _nms_call(heat)
    sc, fl, po = _get_sc_select()(z.reshape(_B * _NSEG, 128),
                                  m.reshape(_B, _NSEG),
                                  pred_regression.reshape(-1))
    pois = [po[:, r, :] for r in range(8)]
    out14 = _decode_call(sc, fl, pois, trans_mat.reshape(_B, 9),
                         Kmat.reshape(_B, 9),
                         img_size.astype(jnp.float32))[0]
    return jnp.transpose(out14, (1, 2, 0))[:, :_K, :].reshape(_B * _K, 14)


# speedup vs baseline: 1.1537x; 1.0973x over previous
import functools

import jax
import jax.numpy as jnp
import numpy as np
from jax import lax
from jax.experimental import pallas as pl
from jax.experimental.pallas import tpu as pltpu
from jax.experimental.pallas import tpu_sc as plsc

_PI = float(np.pi)
_DEPTH0, _DEPTH1 = 28.01, 16.32
_DIMS_TBL = ((3.88, 1.63, 1.53), (1.76, 1.73, 0.6), (0.84, 1.76, 0.66))
_THR = 0.25
_K = 100
_KPAD = 128

_B, _C, _H, _W = 8, 3, 192, 640
_HW = _H * _W
_NSEG = _C * _H * (_W // 128)
_PMAX = 256
_TARGET = 128
_QVAL = float(np.nextafter(np.float32(0.25), np.float32(1.0)))

_SX = (-1.0, 1.0, 1.0, 1.0, 1.0, -1.0, -1.0, -1.0)
_SY = (-1.0, -1.0, 0.0, 0.0, -1.0, -1.0, 0.0, 0.0)
_SZ = (-1.0, -1.0, -1.0, 1.0, 1.0, 1.0, 1.0, -1.0)


def _nms_body(h_ref, z_ref, m_ref):
    x = h_ref[0]
    ncol = jnp.full((_H, 1), -1.0, jnp.float32)
    left = jnp.concatenate([x[:, 1:], ncol], axis=1)
    right = jnp.concatenate([ncol, x[:, :-1]], axis=1)
    rm = jnp.maximum(jnp.maximum(left, right), x)
    nrow = jnp.full((1, _W), -1.0, jnp.float32)
    up = jnp.concatenate([rm[1:, :], nrow], axis=0)
    dn = jnp.concatenate([nrow, rm[:-1, :]], axis=0)
    hm = jnp.maximum(jnp.maximum(up, dn), rm)
    z = jnp.where(hm == x, x, 0.0)
    z_ref[...] = z.reshape(_H * (_W // 128), 128)
    cols = [jnp.max(z[:, i * 128:(i + 1) * 128], axis=1, keepdims=True)
            for i in range(_W // 128)]
    m_ref[0] = jnp.concatenate(cols, axis=1)


def _nms_call(heat, interpret=False):
    return pl.pallas_call(
        _nms_body,
        grid=(_B * _C,),
        compiler_params=pltpu.CompilerParams(
            dimension_semantics=("parallel",)),
        in_specs=[pl.BlockSpec((1, _H, _W), lambda i: (i, 0, 0))],
        out_specs=[
            pl.BlockSpec((_H * (_W // 128), 128), lambda i: (i, 0)),
            pl.BlockSpec((1, _H, _W // 128), lambda i: (i, 0, 0)),
        ],
        out_shape=[
            jax.ShapeDtypeStruct((_B * _NSEG, 128), jnp.float32),
            jax.ShapeDtypeStruct((_B * _C, _H, _W // 128), jnp.float32),
        ],
        interpret=interpret,
    )(heat)


def _sc_body(z_hbm, m_hbm, reg_hbm, sc_out, fl_out, po_out,
             m_v, segid_v, cmax_v, seg_v, os_v, of_v, gi_v, pv_v, sem):
    nc = 2
    wid = lax.axis_index("s") * nc + lax.axis_index("c")

    @pl.when(wid < _B)
    def _worker():
        b = wid
        pltpu.sync_copy(m_hbm.at[b], m_v)

        zero16i = jnp.zeros((16,), jnp.int32)
        zero16f = jnp.zeros((16,), jnp.float32)
        neg16 = jnp.full((16,), -1.0, jnp.float32)
        lane = lax.iota(jnp.int32, 16)

        def init_body(c, _):
            cmax_v[pl.ds(c * 16, 16)] = neg16
            segid_v[pl.ds(c * 16, 16)] = zero16i
            return 0

        lax.fori_loop(0, _PMAX // 16, init_body, 0)

        def init2_body(c, _):
            os_v[pl.ds(c * 16, 16)] = zero16f
            of_v[pl.ds(c * 16, 16)] = zero16i
            return 0

        lax.fori_loop(0, _KPAD // 16, init2_body, 0)

        nchunk = _NSEG // 16

        def _count_ge(t):
            def body(c, acc):
                for u in range(10):
                    v = m_v[pl.ds(c * 160 + u * 16, 16)]
                    acc = acc + jnp.where(v >= t, 1, 0)
                return acc
            accv = lax.fori_loop(0, nchunk // 10, body, zero16i)
            return jnp.sum(accv)

        def bis(_, lohi):
            lo, hi = lohi
            mid = (lo + hi) * 0.5
            cnt = _count_ge(mid)
            big = cnt >= _TARGET
            return jnp.where(big, mid, lo), jnp.where(big, hi, mid)

        lo, hi = lax.fori_loop(0, 18, bis, (jnp.float32(_QVAL),
                                            jnp.float32(1.0)))
        tval = lo

        def comp_body(c, cur):
            v = m_v[pl.ds(c * 16, 16)]
            msk = v >= tval
            n = jnp.max(plsc.all_reduce_population_count(msk))

            @pl.when(cur <= _PMAX - 16)
            def _():
                ids = lane + c * 16
                plsc.store_compressed(segid_v.at[pl.ds(cur, 16)], ids,
                                      mask=msk)
                plsc.store_compressed(cmax_v.at[pl.ds(cur, 16)], v, mask=msk)

            return jnp.where(cur <= _PMAX - 16, cur + n, cur)

        ncand = lax.fori_loop(0, nchunk, comp_body, jnp.int32(0))

        base_row = b * _NSEG

        def gidx_body(c, _):
            gi_v[c // 8, pl.ds((c % 8) * 16, 16)] = (
                segid_v[pl.ds(c * 16, 16)] + base_row)
            return 0

        lax.fori_loop(0, _PMAX // 16, gidx_body, 0)
        cp0 = pltpu.async_copy(z_hbm.at[gi_v.at[0]], seg_v.at[pl.ds(0, 128)],
                               sem)

        @pl.when(ncand > 128)
        def _extra_gather():
            pltpu.async_copy(z_hbm.at[gi_v.at[1]],
                             seg_v.at[pl.ds(128, 128)], sem).wait()

        cp0.wait()

        def _extract_i(ref, i):
            chunk = ref[pl.ds((i // 16) * 16, 16)]
            return jnp.max(jnp.where(lane == (i % 16), chunk, 0))

        def hier_body(c, h):
            return jnp.where(lane == c, jnp.max(cmax_v[pl.ds(c * 16, 16)]), h)

        c2_init = lax.fori_loop(0, _PMAX // 16, hier_body, neg16)

        def pop_body(k, c2):
            m = jnp.max(c2)
            c = jnp.min(jnp.where(c2 == m, lane, 9999))
            v = cmax_v[pl.ds(c * 16, 16)]
            sl = jnp.min(jnp.where(v == m, lane, 9999))
            si = c * 16 + sl

            jv = jnp.full((16,), 9999, jnp.int32)
            for u in range(8):
                sv = seg_v[si, pl.ds(u * 16, 16)]
                jv = jnp.minimum(jv, jnp.where(sv == m, lane + u * 16, 9999))
            j = jnp.minimum(jnp.min(jv), 127)
            sid = _extract_i(segid_v, si)
            flat = sid * 128 + j
            kc = (k // 16) * 16
            kl = k % 16
            os_v[pl.ds(kc, 16)] = jnp.where(lane == kl, m,
                                            os_v[pl.ds(kc, 16)])
            of_v[pl.ds(kc, 16)] = jnp.where(lane == kl, flat,
                                            of_v[pl.ds(kc, 16)])
            jc = j // 16
            jl = j % 16
            nms = neg16
            for u in range(8):
                sv = seg_v[si, pl.ds(u * 16, 16)]
                sv = jnp.where((jc == u) & (lane == jl), -1.0, sv)
                nms = jnp.maximum(nms, sv)
            nm = jnp.max(nms)
            seg_v[si, pl.ds(jc * 16, 16)] = jnp.where(
                lane == jl, -1.0, seg_v[si, pl.ds(jc * 16, 16)])
            nv = jnp.where(lane == sl, nm, v)
            cmax_v[pl.ds(c * 16, 16)] = nv
            return jnp.where(lane == c, jnp.max(nv), c2)

        lax.fori_loop(0, _K, pop_body, c2_init)

        rbase = b * (8 * _HW)

        def ridx_body(p, _):
            r = p // 8
            c = p % 8
            sp = of_v[pl.ds(c * 16, 16)] % _HW
            gi_v[r, pl.ds(c * 16, 16)] = sp + (rbase + r * _HW)
            return 0

        lax.fori_loop(0, 64, ridx_body, 0)
        cps = [pltpu.async_copy(reg_hbm.at[gi_v.at[r]], pv_v.at[r], sem)
               for r in range(8)]
        for cp in cps:
            cp.wait()

        pltpu.sync_copy(os_v, sc_out.at[b])
        pltpu.sync_copy(of_v, fl_out.at[b])
        pltpu.sync_copy(pv_v, po_out.at[b])


@functools.lru_cache(maxsize=None)
def _get_sc_select():
  return functools.partial(
    pl.kernel,
    mesh=plsc.VectorSubcoreMesh(core_axis_name="c", subcore_axis_name="s"),
    compiler_params=pltpu.CompilerParams(needs_layout_passes=False),
    out_type=[
        jax.ShapeDtypeStruct((_B, _KPAD), jnp.float32),
        jax.ShapeDtypeStruct((_B, _KPAD), jnp.int32),
        jax.ShapeDtypeStruct((_B, 8, _KPAD), jnp.float32),
    ],
    scratch_types=[
        pltpu.VMEM((_NSEG,), jnp.float32),
        pltpu.VMEM((_PMAX,), jnp.int32),
        pltpu.VMEM((_PMAX,), jnp.float32),
        pltpu.VMEM((_PMAX, 128), jnp.float32),
        pltpu.VMEM((_KPAD,), jnp.float32),
        pltpu.VMEM((_KPAD,), jnp.int32),
        pltpu.VMEM((8, 128), jnp.int32),
        pltpu.VMEM((8, 128), jnp.float32),
        pltpu.SemaphoreType.DMA,
    ],
  )(_sc_body)


def _inv3(m9):
    a, bb, cc = m9[:, 0:1], m9[:, 1:2], m9[:, 2:3]
    d, e, f = m9[:, 3:4], m9[:, 4:5], m9[:, 5:6]
    g, h, i = m9[:, 6:7], m9[:, 7:8], m9[:, 8:9]
    A = e * i - f * h
    Bc = -(d * i - f * g)
    Cc = d * h - e * g
    det = a * A + bb * Bc + cc * Cc
    r = 1.0 / det
    return ((A * r, -(bb * i - cc * h) * r, (bb * f - cc * e) * r),
            (Bc * r, (a * i - cc * g) * r, -(a * f - cc * d) * r),
            (Cc * r, -(a * h - bb * g) * r, (a * e - bb * d) * r))


def _decode_body(s_ref, f_ref, p0r, p1r, p2r, p3r, p4r, p5r, p6r, p7r,
                 t_ref, k_ref, i_ref, o_ref):
    score = s_ref[...]
    flat = f_ref[...]
    cls = flat // _HW
    sp = flat - cls * _HW
    ysi = sp // _W
    xsi = sp - ysi * _W
    xs = xsi.astype(jnp.float32)
    ys = ysi.astype(jnp.float32)
    clsf = cls.astype(jnp.float32)

    depth = p0r[...] * _DEPTH1 + _DEPTH0
    px = xs + p1r[...]
    py = ys + p2r[...]

    ti = _inv3(t_ref[...])
    ix = (ti[0][0] * px + ti[0][1] * py + ti[0][2]) * depth
    iy = (ti[1][0] * px + ti[1][1] * py + ti[1][2]) * depth
    iz = (ti[2][0] * px + ti[2][1] * py + ti[2][2]) * depth
    ki = _inv3(k_ref[...])
    lx = ki[0][0] * ix + ki[0][1] * iy + ki[0][2] * iz
    ly = ki[1][0] * ix + ki[1][1] * iy + ki[1][2] * iz
    lz = ki[2][0] * ix + ki[2][1] * iy + ki[2][2] * iz

    is0 = jnp.where(cls == 0, 1.0, 0.0)
    is1 = jnp.where(cls == 1, 1.0, 0.0)
    is2 = 1.0 - is0 - is1
    d0 = jnp.exp(p3r[...]) * (is0 * _DIMS_TBL[0][0] + is1 * _DIMS_TBL[1][0]
                              + is2 * _DIMS_TBL[2][0])
    d1 = jnp.exp(p4r[...]) * (is0 * _DIMS_TBL[0][1] + is1 * _DIMS_TBL[1][1]
                              + is2 * _DIMS_TBL[2][1])
    d2 = jnp.exp(p5r[...]) * (is0 * _DIMS_TBL[0][2] + is1 * _DIMS_TBL[1][2]
                              + is2 * _DIMS_TBL[2][2])
    ly = ly + d1 * 0.5

    one = jnp.ones_like(lx)
    rays = jnp.arctan2(lx / (lz + 1e-7), one)
    ori0, ori1 = p6r[...], p7r[...]
    a0 = jnp.arctan2(ori0 / (ori1 + 1e-7), one)
    alpha = jnp.where(ori1 >= 0, a0 - _PI / 2.0, a0 + _PI / 2.0)
    roty = alpha + rays
    roty = jnp.where(roty > _PI, roty - 2.0 * _PI, roty)
    roty = jnp.where(roty < -_PI, roty + 2.0 * _PI, roty)

    cr = jnp.cos(roty)
    sr = jnp.sin(roty)
    kk = k_ref[...]
    k00, k01, k02 = kk[:, 0:1], kk[:, 1:2], kk[:, 2:3]
    k10, k11, k12 = kk[:, 3:4], kk[:, 4:5], kk[:, 5:6]
    k20, k21, k22 = kk[:, 6:7], kk[:, 7:8], kk[:, 8:9]

    big = jnp.float32(1e30)
    umin = jnp.full_like(score, big)
    umax = jnp.full_like(score, -big)
    vmin = jnp.full_like(score, big)
    vmax = jnp.full_like(score, -big)
    for t in range(8):
        cx = d0 * (0.5 * _SX[t])
        cy = d1 * _SY[t]
        cz = d2 * (0.5 * _SZ[t])
        X = cr * cx + sr * cz + lx
        Y = cy + ly
        Zc = -sr * cx + cr * cz + lz
        w_ = k20 * X + k21 * Y + k22 * Zc
        u_ = (k00 * X + k01 * Y + k02 * Zc) / w_
        v_ = (k10 * X + k11 * Y + k12 * Zc) / w_
        umin = jnp.minimum(umin, u_)
        umax = jnp.maximum(umax, u_)
        vmin = jnp.minimum(vmin, v_)
        vmax = jnp.maximum(vmax, v_)

    iw = i_ref[0:1, 0:1]
    ih = i_ref[0:1, 1:2]
    xmin = jnp.clip(umin, 0.0, iw)
    xmax = jnp.clip(umax, 0.0, iw)
    ymin = jnp.clip(vmin, 0.0, ih)
    ymax = jnp.clip(vmax, 0.0, ih)

    keep = score > _THR
    rows = (clsf, alpha, xmin, ymin, xmax, ymax, d1, d2, d0,
            lx, ly, lz, roty, score)
    for idx, rr in enumerate(rows):
        o_ref[idx] = jnp.where(keep, rr, 0.0)


def _decode_call(sc, fl, pois, t9, k9, isz, interpret=False):
    full = lambda s: pl.BlockSpec(s, lambda: tuple(0 for _ in s))
    return pl.pallas_call(
        _decode_body,
        in_specs=[full((_B, _KPAD)), full((_B, _KPAD))]
        + [full((_B, _KPAD))] * 8
        + [full((_B, 9)), full((_B, 9)), full((_B, 2))],
        out_specs=[full((14, _B, _KPAD))],
        out_shape=[jax.ShapeDtypeStruct((14, _B, _KPAD), jnp.float32)],
        interpret=interpret,
    )(sc, fl, *pois, t9, k9, isz)


def kernel(pred_heatmap, pred_regression, trans_mat, Kmat, img_size):
    heat = pred_heatmap.reshape(_B * _C, _H, _W)
    z, m = _nms_call(heat)
    sc, fl, po = _get_sc_select()(z, m.reshape(_B, _NSEG),
                                  pred_regression.reshape(-1))
    pois = [po[:, r, :] for r in range(8)]
    out14 = _decode_call(sc, fl, pois, trans_mat.reshape(_B, 9),
                         Kmat.reshape(_B, 9),
                         img_size.astype(jnp.float32))[0]
    return jnp.transpose(out14, (1, 2, 0))[:, :_K, :].reshape(_B * _K, 14)


# speedup vs baseline: 1.1657x; 1.0105x over previous
import functools

import jax
import jax.numpy as jnp
import numpy as np
from jax import lax
from jax.experimental import pallas as pl
from jax.experimental.pallas import tpu as pltpu
from jax.experimental.pallas import tpu_sc as plsc

_PI = float(np.pi)
_DEPTH0, _DEPTH1 = 28.01, 16.32
_DIMS_TBL = ((3.88, 1.63, 1.53), (1.76, 1.73, 0.6), (0.84, 1.76, 0.66))
_THR = 0.25
_K = 100
_KPAD = 128

_B, _C, _H, _W = 8, 3, 192, 640
_HW = _H * _W
_NSEG = _C * _H * (_W // 128)
_PMAX = 256
_TARGET = 128
_QVAL = float(np.nextafter(np.float32(0.25), np.float32(1.0)))

_SX = (-1.0, 1.0, 1.0, 1.0, 1.0, -1.0, -1.0, -1.0)
_SY = (-1.0, -1.0, 0.0, 0.0, -1.0, -1.0, 0.0, 0.0)
_SZ = (-1.0, -1.0, -1.0, 1.0, 1.0, 1.0, 1.0, -1.0)


def _nms_body(h_ref, z_ref, m_ref):
    x = h_ref[0]
    ncol = jnp.full((_H, 1), -1.0, jnp.float32)
    left = jnp.concatenate([x[:, 1:], ncol], axis=1)
    right = jnp.concatenate([ncol, x[:, :-1]], axis=1)
    rm = jnp.maximum(jnp.maximum(left, right), x)
    nrow = jnp.full((1, _W), -1.0, jnp.float32)
    up = jnp.concatenate([rm[1:, :], nrow], axis=0)
    dn = jnp.concatenate([nrow, rm[:-1, :]], axis=0)
    hm = jnp.maximum(jnp.maximum(up, dn), rm)
    z = jnp.where(hm == x, x, 0.0)
    z_ref[...] = z.reshape(_H * (_W // 128), 128)
    cols = [jnp.max(z[:, i * 128:(i + 1) * 128], axis=1, keepdims=True)
            for i in range(_W // 128)]
    m_ref[0] = jnp.concatenate(cols, axis=1)


def _nms_call(heat, interpret=False):
    return pl.pallas_call(
        _nms_body,
        grid=(_B * _C,),
        compiler_params=pltpu.CompilerParams(
            dimension_semantics=("parallel",)),
        in_specs=[pl.BlockSpec((1, _H, _W), lambda i: (i, 0, 0))],
        out_specs=[
            pl.BlockSpec((_H * (_W // 128), 128), lambda i: (i, 0)),
            pl.BlockSpec((1, _H, _W // 128), lambda i: (i, 0, 0)),
        ],
        out_shape=[
            jax.ShapeDtypeStruct((_B * _NSEG, 128), jnp.float32),
            jax.ShapeDtypeStruct((_B * _C, _H, _W // 128), jnp.float32),
        ],
        interpret=interpret,
    )(heat)


def _sc_body(z_hbm, m_hbm, reg_hbm, sc_out, fl_out, po_out,
             m_v, segid_v, cmax_v, seg_v, os_v, of_v, gi_v, pv_v, sem):
    nc = 2
    wid = lax.axis_index("s") * nc + lax.axis_index("c")

    @pl.when(wid < _B)
    def _worker():
        b = wid
        pltpu.sync_copy(m_hbm.at[b], m_v)

        zero16i = jnp.zeros((16,), jnp.int32)
        zero16f = jnp.zeros((16,), jnp.float32)
        neg16 = jnp.full((16,), -1.0, jnp.float32)
        lane = lax.iota(jnp.int32, 16)

        def init_body(c, _):
            cmax_v[pl.ds(c * 16, 16)] = neg16
            segid_v[pl.ds(c * 16, 16)] = zero16i
            return 0

        lax.fori_loop(0, _PMAX // 16, init_body, 0)

        def init2_body(c, _):
            os_v[pl.ds(c * 16, 16)] = zero16f
            of_v[pl.ds(c * 16, 16)] = zero16i
            return 0

        lax.fori_loop(0, _KPAD // 16, init2_body, 0)

        nchunk = _NSEG // 16

        def _count_ge(t):
            def body(c, acc):
                for u in range(10):
                    v = m_v[pl.ds(c * 160 + u * 16, 16)]
                    acc = acc + jnp.where(v >= t, 1, 0)
                return acc
            accv = lax.fori_loop(0, nchunk // 10, body, zero16i)
            return jnp.sum(accv)

        def bis(_, lohi):
            lo, hi = lohi
            mid = (lo + hi) * 0.5
            cnt = _count_ge(mid)
            big = cnt >= _TARGET
            return jnp.where(big, mid, lo), jnp.where(big, hi, mid)

        lo, hi = lax.fori_loop(0, 18, bis, (jnp.float32(_QVAL),
                                            jnp.float32(1.0)))
        tval = lo

        def comp_body(c, cur):
            for u in range(2):
                v = m_v[pl.ds(c * 32 + u * 16, 16)]
                msk = v >= tval
                n = jnp.max(plsc.all_reduce_population_count(msk))

                @pl.when(cur <= _PMAX - 16)
                def _():
                    ids = lane + (c * 32 + u * 16)
                    plsc.store_compressed(segid_v.at[pl.ds(cur, 16)], ids,
                                          mask=msk)
                    plsc.store_compressed(cmax_v.at[pl.ds(cur, 16)], v,
                                          mask=msk)

                cur = jnp.where(cur <= _PMAX - 16, cur + n, cur)
            return cur

        ncand = lax.fori_loop(0, nchunk // 2, comp_body, jnp.int32(0))

        base_row = b * _NSEG

        def gidx_body(c, _):
            gi_v[c // 8, pl.ds((c % 8) * 16, 16)] = (
                segid_v[pl.ds(c * 16, 16)] + base_row)
            return 0

        lax.fori_loop(0, _PMAX // 16, gidx_body, 0)
        cp0 = pltpu.async_copy(z_hbm.at[gi_v.at[0]], seg_v.at[pl.ds(0, 128)],
                               sem)

        @pl.when(ncand > 128)
        def _extra_gather():
            pltpu.async_copy(z_hbm.at[gi_v.at[1]],
                             seg_v.at[pl.ds(128, 128)], sem).wait()

        cp0.wait()

        def _extract_i(ref, i):
            chunk = ref[pl.ds((i // 16) * 16, 16)]
            return jnp.max(jnp.where(lane == (i % 16), chunk, 0))

        def hier_body(c, h):
            return jnp.where(lane == c, jnp.max(cmax_v[pl.ds(c * 16, 16)]), h)

        c2_init = lax.fori_loop(0, _PMAX // 16, hier_body, neg16)

        def pop_body(k, c2):
            m = jnp.max(c2)
            c = jnp.min(jnp.where(c2 == m, lane, 9999))
            v = cmax_v[pl.ds(c * 16, 16)]
            sl = jnp.min(jnp.where(v == m, lane, 9999))
            si = c * 16 + sl

            jv = jnp.full((16,), 9999, jnp.int32)
            for u in range(8):
                sv = seg_v[si, pl.ds(u * 16, 16)]
                jv = jnp.minimum(jv, jnp.where(sv == m, lane + u * 16, 9999))
            j = jnp.minimum(jnp.min(jv), 127)
            sid = _extract_i(segid_v, si)
            flat = sid * 128 + j
            kc = (k // 16) * 16
            kl = k % 16
            os_v[pl.ds(kc, 16)] = jnp.where(lane == kl, m,
                                            os_v[pl.ds(kc, 16)])
            of_v[pl.ds(kc, 16)] = jnp.where(lane == kl, flat,
                                            of_v[pl.ds(kc, 16)])
            jc = j // 16
            jl = j % 16
            nms = neg16
            for u in range(8):
                sv = seg_v[si, pl.ds(u * 16, 16)]
                sv = jnp.where((jc == u) & (lane == jl), -1.0, sv)
                nms = jnp.maximum(nms, sv)
            nm = jnp.max(nms)
            seg_v[si, pl.ds(jc * 16, 16)] = jnp.where(
                lane == jl, -1.0, seg_v[si, pl.ds(jc * 16, 16)])
            nv = jnp.where(lane == sl, nm, v)
            cmax_v[pl.ds(c * 16, 16)] = nv
            return jnp.where(lane == c, jnp.max(nv), c2)

        lax.fori_loop(0, _K, pop_body, c2_init)

        rbase = b * (8 * _HW)

        def ridx_body(p, _):
            r = p // 8
            c = p % 8
            sp = of_v[pl.ds(c * 16, 16)] % _HW
            gi_v[r, pl.ds(c * 16, 16)] = sp + (rbase + r * _HW)
            return 0

        lax.fori_loop(0, 64, ridx_body, 0)
        cps = [pltpu.async_copy(reg_hbm.at[gi_v.at[r]], pv_v.at[r], sem)
               for r in range(8)]
        for cp in cps:
            cp.wait()

        pltpu.sync_copy(os_v, sc_out.at[b])
        pltpu.sync_copy(of_v, fl_out.at[b])
        pltpu.sync_copy(pv_v, po_out.at[b])


@functools.lru_cache(maxsize=None)
def _get_sc_select():
  return functools.partial(
    pl.kernel,
    mesh=plsc.VectorSubcoreMesh(core_axis_name="c", subcore_axis_name="s"),
    compiler_params=pltpu.CompilerParams(needs_layout_passes=False),
    out_type=[
        jax.ShapeDtypeStruct((_B, _KPAD), jnp.float32),
        jax.ShapeDtypeStruct((_B, _KPAD), jnp.int32),
        jax.ShapeDtypeStruct((_B, 8, _KPAD), jnp.float32),
    ],
    scratch_types=[
        pltpu.VMEM((_NSEG,), jnp.float32),
        pltpu.VMEM((_PMAX,), jnp.int32),
        pltpu.VMEM((_PMAX,), jnp.float32),
        pltpu.VMEM((_PMAX, 128), jnp.float32),
        pltpu.VMEM((_KPAD,), jnp.float32),
        pltpu.VMEM((_KPAD,), jnp.int32),
        pltpu.VMEM((8, 128), jnp.int32),
        pltpu.VMEM((8, 128), jnp.float32),
        pltpu.SemaphoreType.DMA,
    ],
  )(_sc_body)


def _inv3(m9):
    a, bb, cc = m9[:, 0:1], m9[:, 1:2], m9[:, 2:3]
    d, e, f = m9[:, 3:4], m9[:, 4:5], m9[:, 5:6]
    g, h, i = m9[:, 6:7], m9[:, 7:8], m9[:, 8:9]
    A = e * i - f * h
    Bc = -(d * i - f * g)
    Cc = d * h - e * g
    det = a * A + bb * Bc + cc * Cc
    r = 1.0 / det
    return ((A * r, -(bb * i - cc * h) * r, (bb * f - cc * e) * r),
            (Bc * r, (a * i - cc * g) * r, -(a * f - cc * d) * r),
            (Cc * r, -(a * h - bb * g) * r, (a * e - bb * d) * r))


def _decode_body(s_ref, f_ref, p_ref, t_ref, k_ref, i_ref, o_ref):
    p0r, p1r, p2r, p3r, p4r, p5r, p6r, p7r = (
        p_ref.at[:, r, :] for r in range(8))
    score = s_ref[...]
    flat = f_ref[...]
    cls = flat // _HW
    sp = flat - cls * _HW
    ysi = sp // _W
    xsi = sp - ysi * _W
    xs = xsi.astype(jnp.float32)
    ys = ysi.astype(jnp.float32)
    clsf = cls.astype(jnp.float32)

    depth = p0r[...] * _DEPTH1 + _DEPTH0
    px = xs + p1r[...]
    py = ys + p2r[...]

    ti = _inv3(t_ref[...])
    ix = (ti[0][0] * px + ti[0][1] * py + ti[0][2]) * depth
    iy = (ti[1][0] * px + ti[1][1] * py + ti[1][2]) * depth
    iz = (ti[2][0] * px + ti[2][1] * py + ti[2][2]) * depth
    ki = _inv3(k_ref[...])
    lx = ki[0][0] * ix + ki[0][1] * iy + ki[0][2] * iz
    ly = ki[1][0] * ix + ki[1][1] * iy + ki[1][2] * iz
    lz = ki[2][0] * ix + ki[2][1] * iy + ki[2][2] * iz

    is0 = jnp.where(cls == 0, 1.0, 0.0)
    is1 = jnp.where(cls == 1, 1.0, 0.0)
    is2 = 1.0 - is0 - is1
    d0 = jnp.exp(p3r[...]) * (is0 * _DIMS_TBL[0][0] + is1 * _DIMS_TBL[1][0]
                              + is2 * _DIMS_TBL[2][0])
    d1 = jnp.exp(p4r[...]) * (is0 * _DIMS_TBL[0][1] + is1 * _DIMS_TBL[1][1]
                              + is2 * _DIMS_TBL[2][1])
    d2 = jnp.exp(p5r[...]) * (is0 * _DIMS_TBL[0][2] + is1 * _DIMS_TBL[1][2]
                              + is2 * _DIMS_TBL[2][2])
    ly = ly + d1 * 0.5

    one = jnp.ones_like(lx)
    rays = jnp.arctan2(lx / (lz + 1e-7), one)
    ori0, ori1 = p6r[...], p7r[...]
    a0 = jnp.arctan2(ori0 / (ori1 + 1e-7), one)
    alpha = jnp.where(ori1 >= 0, a0 - _PI / 2.0, a0 + _PI / 2.0)
    roty = alpha + rays
    roty = jnp.where(roty > _PI, roty - 2.0 * _PI, roty)
    roty = jnp.where(roty < -_PI, roty + 2.0 * _PI, roty)

    cr = jnp.cos(roty)
    sr = jnp.sin(roty)
    kk = k_ref[...]
    k00, k01, k02 = kk[:, 0:1], kk[:, 1:2], kk[:, 2:3]
    k10, k11, k12 = kk[:, 3:4], kk[:, 4:5], kk[:, 5:6]
    k20, k21, k22 = kk[:, 6:7], kk[:, 7:8], kk[:, 8:9]

    big = jnp.float32(1e30)
    umin = jnp.full_like(score, big)
    umax = jnp.full_like(score, -big)
    vmin = jnp.full_like(score, big)
    vmax = jnp.full_like(score, -big)
    for t in range(8):
        cx = d0 * (0.5 * _SX[t])
        cy = d1 * _SY[t]
        cz = d2 * (0.5 * _SZ[t])
        X = cr * cx + sr * cz + lx
        Y = cy + ly
        Zc = -sr * cx + cr * cz + lz
        w_ = k20 * X + k21 * Y + k22 * Zc
        u_ = (k00 * X + k01 * Y + k02 * Zc) / w_
        v_ = (k10 * X + k11 * Y + k12 * Zc) / w_
        umin = jnp.minimum(umin, u_)
        umax = jnp.maximum(umax, u_)
        vmin = jnp.minimum(vmin, v_)
        vmax = jnp.maximum(vmax, v_)

    iw = i_ref[0:1, 0:1]
    ih = i_ref[0:1, 1:2]
    xmin = jnp.clip(umin, 0.0, iw)
    xmax = jnp.clip(umax, 0.0, iw)
    ymin = jnp.clip(vmin, 0.0, ih)
    ymax = jnp.clip(vmax, 0.0, ih)

    keep = score > _THR
    rows = (clsf, alpha, xmin, ymin, xmax, ymax, d1, d2, d0,
            lx, ly, lz, roty, score)
    for idx, rr in enumerate(rows):
        o_ref[idx] = jnp.where(keep, rr, 0.0)


def _decode_call(sc, fl, po, t9, k9, isz, interpret=False):
    full = lambda s: pl.BlockSpec(s, lambda: tuple(0 for _ in s))
    return pl.pallas_call(
        _decode_body,
        in_specs=[full((_B, _KPAD)), full((_B, _KPAD)),
                  full((_B, 8, _KPAD)),
                  full((_B, 9)), full((_B, 9)), full((_B, 2))],
        out_specs=[full((14, _B, _KPAD))],
        out_shape=[jax.ShapeDtypeStruct((14, _B, _KPAD), jnp.float32)],
        interpret=interpret,
    )(sc, fl, po, t9, k9, isz)


def kernel(pred_heatmap, pred_regression, trans_mat, Kmat, img_size):
    heat = pred_heatmap.reshape(_B * _C, _H, _W)
    z, m = _nms_call(heat)
    sc, fl, po = _get_sc_select()(z, m.reshape(_B, _NSEG),
                                  pred_regression.reshape(-1))
    out14 = _decode_call(sc, fl, po, trans_mat.reshape(_B, 9),
                         Kmat.reshape(_B, 9),
                         img_size.astype(jnp.float32))[0]
    return jnp.transpose(out14, (1, 2, 0))[:, :_K, :].reshape(_B * _K, 14)
